# baseline (speedup 1.0000x reference)
import functools

import jax
import jax.numpy as jnp
from jax import lax
from jax.experimental import pallas as pl
from jax.experimental.pallas import tpu as pltpu
from jax.experimental.pallas import tpu_sc as plsc

N = 50000
E = 1600000
D = 32
DE = 38

NC = 2
NS = 16
L = 16

NP = 50176
ROWS_PER_TILE = NP // NS
ZCH = 56

CH = 100
NCHUNK = E // CH
TILE_CH = NCHUNK // (NC * NS)
NB = 3
ND = 2 * NB

_mesh = plsc.VectorSubcoreMesh(core_axis_name="c", subcore_axis_name="s")
_sc_params = pltpu.CompilerParams(use_tc_tiling_on_sc=False)


def _zero_rows(zb, width):
    def zrow(i, _):
        for h in range(width // L):
            zb[i, pl.ds(h * L, L)] = jnp.zeros((L,), jnp.float32)
        return 0

    lax.fori_loop(0, CH, zrow, 0)


def _zero_shared(zb, shared):
    s = lax.axis_index("s")
    r0 = s * ROWS_PER_TILE

    def zcp(k, _):
        pltpu.sync_copy(zb, shared.at[pl.ds(r0 + k * ZCH, ZCH), :])
        return 0

    lax.fori_loop(0, ROWS_PER_TILE // ZCH, zcp, 0)


def _make_sc_edge(off):
  @functools.partial(
    pl.kernel,
    out_type=jax.ShapeDtypeStruct((NC, NP, D), jnp.float32),
    mesh=_mesh,
    compiler_params=_sc_params,
    scratch_types=[
        pltpu.VMEM((NB, CH), jnp.int32),
        pltpu.VMEM((ND, CH), jnp.int32),
        pltpu.VMEM((NB, CH, D), jnp.float32),
        pltpu.VMEM((NB, CH, D), jnp.float32),
        pltpu.VMEM((NB, CH, D), jnp.float32),
        pltpu.VMEM_SHARED((NP, D), jnp.float32),
        pltpu.SemaphoreType.DMA((NB,)),
        pltpu.SemaphoreType.DMA((NB,)),
        pltpu.SemaphoreType.DMA((NB,)),
    ],
  )
  def _sc_edge_pass(t_hbm, eaw_hbm, src_hbm, dst_hbm, out_hbm,
                    sidx, didx, rin, eav, rout, s_sh, xsem, isem, osem):
      c = lax.axis_index("c")
      s = lax.axis_index("s")
      _zero_rows(rout.at[0], D)
      _zero_shared(rout.at[0, pl.ds(0, ZCH), :], s_sh)
      plsc.subcore_barrier()

      base = (c * NS + s) * TILE_CH

      def issue_idx(k):
          b = lax.rem(k, NB)
          bd = lax.rem(k, ND)
          pltpu.async_copy(src_hbm.at[base + k], sidx.at[b], xsem.at[b])
          pltpu.async_copy(dst_hbm.at[base + k], didx.at[bd], xsem.at[b])

      def wait_idx(k):
          b = lax.rem(k, NB)
          bd = lax.rem(k, ND)
          pltpu.make_async_copy(src_hbm.at[base + k], sidx.at[b],
                                xsem.at[b]).wait()
          pltpu.make_async_copy(dst_hbm.at[base + k], didx.at[bd],
                                xsem.at[b]).wait()

      def issue_in(k):
          b = lax.rem(k, NB)
          e0 = (base + k) * CH
          pltpu.async_copy(t_hbm.at[sidx.at[b]], rin.at[b], isem.at[b])
          pltpu.async_copy(eaw_hbm.at[pl.ds(e0, CH), pl.ds(off, D)],
                         eav.at[b], isem.at[b])

      def wait_in(k):
          b = lax.rem(k, NB)
          e0 = (base + k) * CH
          pltpu.make_async_copy(t_hbm.at[sidx.at[b]], rin.at[b],
                                isem.at[b]).wait()
          pltpu.make_async_copy(eaw_hbm.at[pl.ds(e0, CH), pl.ds(off, D)],
                                eav.at[b], isem.at[b]).wait()

      def wait_out(b):
          pltpu.make_async_copy(rout.at[b], s_sh.at[didx.at[0]],
                                osem.at[b]).wait()

      issue_idx(0)
      issue_idx(1)
      issue_idx(2)
      wait_idx(0)
      issue_in(0)
      wait_idx(1)
      issue_in(1)

      def step(k, _):
          b = lax.rem(k, NB)

          @pl.when(k >= NB)
          def _():
              wait_out(b)

          @pl.when(k + NB < TILE_CH)
          def _():
              issue_idx(k + NB)

          @pl.when(k + 2 < TILE_CH)
          def _():
              wait_idx(k + 2)
              issue_in(k + 2)

          wait_in(k)

          def inner(j, _):
              for h in range(D // L):
                  sl = pl.ds(h * L, L)
                  v = rin[b, j, sl] + eav[b, j, sl]
                  rout[b, j, sl] = jnp.maximum(v, 0.0)
              return 0

          lax.fori_loop(0, CH, inner, 0, unroll=10)
          pltpu.async_copy(rout.at[b], s_sh.at[didx.at[lax.rem(k, ND)]],
                           osem.at[b], add=True)
          return 0

      lax.fori_loop(0, TILE_CH, step, 0)
      for b in range(NB):
          wait_out(b)
      plsc.subcore_barrier()
      r0 = s * ROWS_PER_TILE
      pltpu.sync_copy(s_sh.at[pl.ds(r0, ROWS_PER_TILE), :],
                      out_hbm.at[c, pl.ds(r0, ROWS_PER_TILE), :])
  return _sc_edge_pass


_sc_edge_layers = [_make_sc_edge(l * D) for l in range(3)]


CW = 16


@functools.partial(
    pl.kernel,
    out_type=jax.ShapeDtypeStruct((NC, NP, CW), jnp.float32),
    mesh=_mesh,
    compiler_params=_sc_params,
    scratch_types=[
        pltpu.VMEM((ND, CH), jnp.int32),
        pltpu.VMEM((CH, CW), jnp.float32),
        pltpu.VMEM((CH, CW), jnp.float32),
        pltpu.VMEM_SHARED((NP, CW), jnp.float32),
        pltpu.SemaphoreType.DMA((NB,)),
        pltpu.SemaphoreType.DMA((NB,)),
    ],
)
def _sc_count(dst_hbm, out_hbm, didx, ones, zb, c_sh, xsem, osem):
    c = lax.axis_index("c")
    s = lax.axis_index("s")

    def orow(i, _):
        ones[i, pl.ds(0, L)] = jnp.ones((L,), jnp.float32)
        return 0

    lax.fori_loop(0, CH, orow, 0)
    _zero_rows(zb, CW)
    _zero_shared(zb.at[pl.ds(0, ZCH), :], c_sh)
    plsc.subcore_barrier()

    base = (c * NS + s) * TILE_CH

    def issue_idx(k):
        b = lax.rem(k, NB)
        bd = lax.rem(k, ND)
        pltpu.async_copy(dst_hbm.at[base + k], didx.at[bd], xsem.at[b])

    def wait_idx(k):
        b = lax.rem(k, NB)
        bd = lax.rem(k, ND)
        pltpu.make_async_copy(dst_hbm.at[base + k], didx.at[bd],
                              xsem.at[b]).wait()

    def wait_out(b):
        pltpu.make_async_copy(ones, c_sh.at[didx.at[0]], osem.at[b]).wait()

    issue_idx(0)
    issue_idx(1)
    issue_idx(2)

    def step(k, _):
        b = lax.rem(k, NB)

        @pl.when(k >= NB)
        def _():
            wait_out(b)

        wait_idx(k)
        pltpu.async_copy(ones, c_sh.at[didx.at[lax.rem(k, ND)]],
                         osem.at[b], add=True)

        @pl.when(k + NB < TILE_CH)
        def _():
            issue_idx(k + NB)
        return 0

    lax.fori_loop(0, TILE_CH, step, 0)
    for b in range(NB):
        wait_out(b)
    plsc.subcore_barrier()
    r0 = s * ROWS_PER_TILE
    pltpu.sync_copy(c_sh.at[pl.ds(r0, ROWS_PER_TILE), :],
                    out_hbm.at[c, pl.ds(r0, ROWS_PER_TILE), :])


BE = 8000


def _edge_mlp(ea, wb, bb):
    def body(ea_ref, w_ref, b_ref, o_ref):
        o_ref[...] = jnp.dot(ea_ref[...], w_ref[...],
                             preferred_element_type=jnp.float32) + b_ref[...]

    return pl.pallas_call(
        body,
        grid=(E // BE,),
        in_specs=[
            pl.BlockSpec((BE, DE), lambda i: (i, 0)),
            pl.BlockSpec((DE, 3 * D), lambda i: (0, 0)),
            pl.BlockSpec((1, 3 * D), lambda i: (0, 0)),
        ],
        out_specs=pl.BlockSpec((BE, 3 * D), lambda i: (i, 0)),
        out_shape=jax.ShapeDtypeStruct((E, 3 * D), jnp.float32),
    )(ea, wb, bb)


BN = 2000


def _node_matmul(xx, w):
    def body(x_ref, w_ref, o_ref):
        o_ref[...] = jnp.dot(x_ref[...], w_ref[...],
                             preferred_element_type=jnp.float32)

    return pl.pallas_call(
        body,
        grid=(N // BN,),
        in_specs=[
            pl.BlockSpec((BN, D), lambda i: (i, 0)),
            pl.BlockSpec((D, D), lambda i: (0, 0)),
        ],
        out_specs=pl.BlockSpec((BN, D), lambda i: (i, 0)),
        out_shape=jax.ShapeDtypeStruct((N, D), jnp.float32),
    )(xx, w)


def _post(s_parts, cnt, w2, b2, w_next):
    def body(s_ref, c_ref, w2_ref, b2_ref, *rest):
        if w_next is not None:
            wn_ref, o_ref = rest
        else:
            (o_ref,) = rest
        ss = s_ref[0] + s_ref[1]
        cn = c_ref[0, :, 0] + c_ref[1, :, 0]
        mm = jnp.dot(ss, w2_ref[...],
                     preferred_element_type=jnp.float32)
        mean = (mm + cn[:, None] * b2_ref[...]) / jnp.maximum(cn, 1.0)[:, None]
        if w_next is not None:
            h = jnp.maximum(mean, 0.0)
            o_ref[...] = jnp.dot(h, wn_ref[...],
                                 preferred_element_type=jnp.float32)
        else:
            o_ref[...] = mean

    in_specs = [
        pl.BlockSpec((NC, BN, D), lambda i: (0, i, 0)),
        pl.BlockSpec((NC, BN, CW), lambda i: (0, i, 0)),
        pl.BlockSpec((D, D), lambda i: (0, 0)),
        pl.BlockSpec((1, D), lambda i: (0, 0)),
    ]
    args = [s_parts, cnt, w2, b2.reshape(1, D)]
    if w_next is not None:
        in_specs.append(pl.BlockSpec((D, D), lambda i: (0, 0)))
        args.append(w_next)
    return pl.pallas_call(
        body,
        grid=(N // BN,),
        in_specs=in_specs,
        out_specs=pl.BlockSpec((BN, D), lambda i: (i, 0)),
        out_shape=jax.ShapeDtypeStruct((N, D), jnp.float32),
    )(*args)


def kernel(x, edge_index, edge_attr,
           W1_1, b1_1, W2_1, b2_1,
           W1_2, b1_2, W2_2, b2_2,
           W1_3, b1_3, W2_3, b2_3):
    src2 = edge_index[0].astype(jnp.int32).reshape(NCHUNK, CH)
    dst2 = edge_index[1].astype(jnp.int32).reshape(NCHUNK, CH)

    wb = jnp.concatenate([W1_1[D:], W1_2[D:], W1_3[D:]], axis=1)
    bb = jnp.concatenate([b1_1, b1_2, b1_3]).reshape(1, 3 * D)
    eaw = _edge_mlp(edge_attr, wb, bb)
    cnt = _sc_count(dst2)

    t = _node_matmul(x, W1_1[:D])
    s_parts = _sc_edge_layers[0](t, eaw, src2, dst2)
    t = _post(s_parts, cnt, W2_1, b2_1, W1_2[:D])
    s_parts = _sc_edge_layers[1](t, eaw, src2, dst2)
    t = _post(s_parts, cnt, W2_2, b2_2, W1_3[:D])
    s_parts = _sc_edge_layers[2](t, eaw, src2, dst2)
    return _post(s_parts, cnt, W2_3, b2_3, None)


# speedup vs baseline: 1.2635x; 1.2635x over previous
import functools

import jax
import jax.numpy as jnp
from jax import lax
from jax.experimental import pallas as pl
from jax.experimental.pallas import tpu as pltpu
from jax.experimental.pallas import tpu_sc as plsc

N = 50000
E = 1600000
D = 32
DE = 38

NC = 2
NS = 16
L = 16

NP = 50176
ROWS_PER_TILE = NP // NS
ZCH = 56

CH = 100
NCHUNK = E // CH
TILE_CH = NCHUNK // (NC * NS)
NB = 3
ND = 2 * NB

_mesh = plsc.VectorSubcoreMesh(core_axis_name="c", subcore_axis_name="s")
_sc_params = pltpu.CompilerParams(use_tc_tiling_on_sc=False)


def _zero_rows(zb, width):
    def zrow(i, _):
        for h in range(width // L):
            zb[i, pl.ds(h * L, L)] = jnp.zeros((L,), jnp.float32)
        return 0

    lax.fori_loop(0, CH, zrow, 0)


def _zero_shared(zb, shared):
    s = lax.axis_index("s")
    r0 = s * ROWS_PER_TILE

    def zcp(k, _):
        pltpu.sync_copy(zb, shared.at[pl.ds(r0 + k * ZCH, ZCH), :])
        return 0

    lax.fori_loop(0, ROWS_PER_TILE // ZCH, zcp, 0)


def _make_sc_edge(off):
  @functools.partial(
    pl.kernel,
    out_type=jax.ShapeDtypeStruct((NC, NP, D), jnp.float32),
    mesh=_mesh,
    compiler_params=_sc_params,
    scratch_types=[
        pltpu.VMEM((NB, CH), jnp.int32),
        pltpu.VMEM((ND, CH), jnp.int32),
        pltpu.VMEM((NB, CH, D), jnp.float32),
        pltpu.VMEM((NB, CH, L), jnp.uint32),
        pltpu.VMEM((NB, CH, D), jnp.float32),
        pltpu.VMEM_SHARED((NP, D), jnp.float32),
        pltpu.SemaphoreType.DMA((NB,)),
        pltpu.SemaphoreType.DMA((NB,)),
        pltpu.SemaphoreType.DMA((NB,)),
    ],
  )
  def _sc_edge_pass(t_hbm, eaw_hbm, src_hbm, dst_hbm, out_hbm,
                    sidx, didx, rin, eav, rout, s_sh, xsem, isem, osem):
      c = lax.axis_index("c")
      s = lax.axis_index("s")
      _zero_rows(rout.at[0], D)
      _zero_shared(rout.at[0, pl.ds(0, ZCH), :], s_sh)
      plsc.subcore_barrier()

      base = (c * NS + s) * TILE_CH

      def issue_idx(k):
          b = lax.rem(k, NB)
          bd = lax.rem(k, ND)
          pltpu.async_copy(src_hbm.at[base + k], sidx.at[b], xsem.at[b])
          pltpu.async_copy(dst_hbm.at[base + k], didx.at[bd], xsem.at[b])

      def wait_idx(k):
          b = lax.rem(k, NB)
          bd = lax.rem(k, ND)
          pltpu.make_async_copy(src_hbm.at[base + k], sidx.at[b],
                                xsem.at[b]).wait()
          pltpu.make_async_copy(dst_hbm.at[base + k], didx.at[bd],
                                xsem.at[b]).wait()

      def issue_in(k):
          b = lax.rem(k, NB)
          e0 = (base + k) * CH
          pltpu.async_copy(t_hbm.at[sidx.at[b]], rin.at[b], isem.at[b])
          pltpu.async_copy(eaw_hbm.at[pl.ds(e0, CH), pl.ds(off, L)],
                         eav.at[b], isem.at[b])

      def wait_in(k):
          b = lax.rem(k, NB)
          e0 = (base + k) * CH
          pltpu.make_async_copy(t_hbm.at[sidx.at[b]], rin.at[b],
                                isem.at[b]).wait()
          pltpu.make_async_copy(eaw_hbm.at[pl.ds(e0, CH), pl.ds(off, L)],
                                eav.at[b], isem.at[b]).wait()

      def wait_out(b):
          pltpu.make_async_copy(rout.at[b], s_sh.at[didx.at[0]],
                                osem.at[b]).wait()

      issue_idx(0)
      issue_idx(1)
      issue_idx(2)
      wait_idx(0)
      issue_in(0)
      wait_idx(1)
      issue_in(1)

      def step(k, _):
          b = lax.rem(k, NB)

          @pl.when(k >= NB)
          def _():
              wait_out(b)

          @pl.when(k + NB < TILE_CH)
          def _():
              issue_idx(k + NB)

          @pl.when(k + 2 < TILE_CH)
          def _():
              wait_idx(k + 2)
              issue_in(k + 2)

          wait_in(k)

          def inner(j, _):
              w = eav[b, j, :]
              e0v = lax.bitcast_convert_type(w << jnp.uint32(16),
                                             jnp.float32)
              e1v = lax.bitcast_convert_type(w & jnp.uint32(0xFFFF0000),
                                             jnp.float32)
              v0 = rin[b, j, pl.ds(0, L)] + e0v
              v1 = rin[b, j, pl.ds(L, L)] + e1v
              rout[b, j, pl.ds(0, L)] = jnp.maximum(v0, 0.0)
              rout[b, j, pl.ds(L, L)] = jnp.maximum(v1, 0.0)
              return 0

          lax.fori_loop(0, CH, inner, 0, unroll=10)
          pltpu.async_copy(rout.at[b], s_sh.at[didx.at[lax.rem(k, ND)]],
                           osem.at[b], add=True)
          return 0

      lax.fori_loop(0, TILE_CH, step, 0)
      for b in range(NB):
          wait_out(b)
      plsc.subcore_barrier()
      r0 = s * ROWS_PER_TILE
      pltpu.sync_copy(s_sh.at[pl.ds(r0, ROWS_PER_TILE), :],
                      out_hbm.at[c, pl.ds(r0, ROWS_PER_TILE), :])
  return _sc_edge_pass


_sc_edge_layers = [_make_sc_edge(l * L) for l in range(3)]


CW = 16


@functools.partial(
    pl.kernel,
    out_type=jax.ShapeDtypeStruct((NC, NP, CW), jnp.float32),
    mesh=_mesh,
    compiler_params=_sc_params,
    scratch_types=[
        pltpu.VMEM((ND, CH), jnp.int32),
        pltpu.VMEM((CH, CW), jnp.float32),
        pltpu.VMEM((CH, CW), jnp.float32),
        pltpu.VMEM_SHARED((NP, CW), jnp.float32),
        pltpu.SemaphoreType.DMA((NB,)),
        pltpu.SemaphoreType.DMA((NB,)),
    ],
)
def _sc_count(dst_hbm, out_hbm, didx, ones, zb, c_sh, xsem, osem):
    c = lax.axis_index("c")
    s = lax.axis_index("s")

    def orow(i, _):
        ones[i, pl.ds(0, L)] = jnp.ones((L,), jnp.float32)
        return 0

    lax.fori_loop(0, CH, orow, 0)
    _zero_rows(zb, CW)
    _zero_shared(zb.at[pl.ds(0, ZCH), :], c_sh)
    plsc.subcore_barrier()

    base = (c * NS + s) * TILE_CH

    def issue_idx(k):
        b = lax.rem(k, NB)
        bd = lax.rem(k, ND)
        pltpu.async_copy(dst_hbm.at[base + k], didx.at[bd], xsem.at[b])

    def wait_idx(k):
        b = lax.rem(k, NB)
        bd = lax.rem(k, ND)
        pltpu.make_async_copy(dst_hbm.at[base + k], didx.at[bd],
                              xsem.at[b]).wait()

    def wait_out(b):
        pltpu.make_async_copy(ones, c_sh.at[didx.at[0]], osem.at[b]).wait()

    issue_idx(0)
    issue_idx(1)
    issue_idx(2)

    def step(k, _):
        b = lax.rem(k, NB)

        @pl.when(k >= NB)
        def _():
            wait_out(b)

        wait_idx(k)
        pltpu.async_copy(ones, c_sh.at[didx.at[lax.rem(k, ND)]],
                         osem.at[b], add=True)

        @pl.when(k + NB < TILE_CH)
        def _():
            issue_idx(k + NB)
        return 0

    lax.fori_loop(0, TILE_CH, step, 0)
    for b in range(NB):
        wait_out(b)
    plsc.subcore_barrier()
    r0 = s * ROWS_PER_TILE
    pltpu.sync_copy(c_sh.at[pl.ds(r0, ROWS_PER_TILE), :],
                    out_hbm.at[c, pl.ds(r0, ROWS_PER_TILE), :])


BE = 8000


def _edge_mlp(ea, wb, bb):
    def body(ea_ref, w_ref, b_ref, o_ref):
        m = jnp.dot(ea_ref[...], w_ref[...],
                    preferred_element_type=jnp.float32) + b_ref[...]
        lo = jnp.concatenate([m[:, 32 * l:32 * l + L] for l in range(3)],
                             axis=1)
        hi = jnp.concatenate([m[:, 32 * l + L:32 * l + 2 * L]
                              for l in range(3)], axis=1)
        lo16 = lax.bitcast_convert_type(lo.astype(jnp.bfloat16), jnp.uint16)
        hi16 = lax.bitcast_convert_type(hi.astype(jnp.bfloat16), jnp.uint16)
        o_ref[...] = (lo16.astype(jnp.uint32)
                      | (hi16.astype(jnp.uint32) << jnp.uint32(16)))

    return pl.pallas_call(
        body,
        grid=(E // BE,),
        in_specs=[
            pl.BlockSpec((BE, DE), lambda i: (i, 0)),
            pl.BlockSpec((DE, 3 * D), lambda i: (0, 0)),
            pl.BlockSpec((1, 3 * D), lambda i: (0, 0)),
        ],
        out_specs=pl.BlockSpec((BE, 3 * D // 2), lambda i: (i, 0)),
        out_shape=jax.ShapeDtypeStruct((E, 3 * D // 2), jnp.uint32),
    )(ea, wb, bb)


BN = 2000


def _node_matmul(xx, w):
    def body(x_ref, w_ref, o_ref):
        o_ref[...] = jnp.dot(x_ref[...], w_ref[...],
                             preferred_element_type=jnp.float32)

    return pl.pallas_call(
        body,
        grid=(N // BN,),
        in_specs=[
            pl.BlockSpec((BN, D), lambda i: (i, 0)),
            pl.BlockSpec((D, D), lambda i: (0, 0)),
        ],
        out_specs=pl.BlockSpec((BN, D), lambda i: (i, 0)),
        out_shape=jax.ShapeDtypeStruct((N, D), jnp.float32),
    )(xx, w)


def _post(s_parts, cnt, w2, b2, w_next):
    def body(s_ref, c_ref, w2_ref, b2_ref, *rest):
        if w_next is not None:
            wn_ref, o_ref = rest
        else:
            (o_ref,) = rest
        ss = s_ref[0] + s_ref[1]
        cn = c_ref[0, :, 0] + c_ref[1, :, 0]
        mm = jnp.dot(ss, w2_ref[...],
                     preferred_element_type=jnp.float32)
        mean = (mm + cn[:, None] * b2_ref[...]) / jnp.maximum(cn, 1.0)[:, None]
        if w_next is not None:
            h = jnp.maximum(mean, 0.0)
            o_ref[...] = jnp.dot(h, wn_ref[...],
                                 preferred_element_type=jnp.float32)
        else:
            o_ref[...] = mean

    in_specs = [
        pl.BlockSpec((NC, BN, D), lambda i: (0, i, 0)),
        pl.BlockSpec((NC, BN, CW), lambda i: (0, i, 0)),
        pl.BlockSpec((D, D), lambda i: (0, 0)),
        pl.BlockSpec((1, D), lambda i: (0, 0)),
    ]
    args = [s_parts, cnt, w2, b2.reshape(1, D)]
    if w_next is not None:
        in_specs.append(pl.BlockSpec((D, D), lambda i: (0, 0)))
        args.append(w_next)
    return pl.pallas_call(
        body,
        grid=(N // BN,),
        in_specs=in_specs,
        out_specs=pl.BlockSpec((BN, D), lambda i: (i, 0)),
        out_shape=jax.ShapeDtypeStruct((N, D), jnp.float32),
    )(*args)


def kernel(x, edge_index, edge_attr,
           W1_1, b1_1, W2_1, b2_1,
           W1_2, b1_2, W2_2, b2_2,
           W1_3, b1_3, W2_3, b2_3):
    src2 = edge_index[0].astype(jnp.int32).reshape(NCHUNK, CH)
    dst2 = edge_index[1].astype(jnp.int32).reshape(NCHUNK, CH)

    wb = jnp.concatenate([W1_1[D:], W1_2[D:], W1_3[D:]], axis=1)
    bb = jnp.concatenate([b1_1, b1_2, b1_3]).reshape(1, 3 * D)
    eaw = _edge_mlp(edge_attr, wb, bb)
    cnt = _sc_count(dst2)

    t = _node_matmul(x, W1_1[:D])
    s_parts = _sc_edge_layers[0](t, eaw, src2, dst2)
    t = _post(s_parts, cnt, W2_1, b2_1, W1_2[:D])
    s_parts = _sc_edge_layers[1](t, eaw, src2, dst2)
    t = _post(s_parts, cnt, W2_2, b2_2, W1_3[:D])
    s_parts = _sc_edge_layers[2](t, eaw, src2, dst2)
    return _post(s_parts, cnt, W2_3, b2_3, None)
